# baseline (speedup 1.0000x reference)
import functools
from collections import OrderedDict

import jax
import jax.numpy as jnp
from jax.experimental import pallas as pl
from jax.experimental.pallas import tpu as pltpu

_BF16 = jnp.bfloat16


def _row_tile(total, target, quantum=8):
    if total <= target:
        return total
    cands = [c for c in range(quantum, target + 1, quantum) if total % c == 0]
    return max(cands) if cands else total


def _c3x3s2_body(xp_ref, w_ref, b_ref, o_ref):
    r = pl.program_id(1)
    _, th, Wo, Cout = o_ref.shape
    r0 = pl.multiple_of(r * th, th)
    bands = (
        xp_ref[0, pl.ds(r0, th), 0, :, :],
        xp_ref[0, pl.ds(r0, th), 1, :, :],
        xp_ref[0, pl.ds(r0 + 1, th), 0, :, :],
    )
    pieces = []
    for band in bands:
        pieces.append(band[:, 0:Wo, :])
        pieces.append(band[:, 1:Wo + 1, :])
    patches = jnp.concatenate(pieces, axis=-1)
    k = patches.shape[-1]
    acc = jnp.dot(patches.reshape(th * Wo, k), w_ref[...],
                  preferred_element_type=jnp.float32)
    acc = jnp.maximum(acc + b_ref[...], 0.0)
    o_ref[...] = acc.reshape(1, th, Wo, Cout).astype(o_ref.dtype)


def _conv3x3_s2_relu_packed(xpp, w, b, row_tile=32):
    N, Hop, _, Wop, Cin2 = xpp.shape
    Cin = Cin2 // 2
    Cout = w.shape[-1]
    Ho, Wo = Hop - 1, Wop - 1
    w4 = jnp.concatenate([w, jnp.zeros((3, 1, Cin, Cout), w.dtype)], axis=1)
    w12 = w4.reshape(12 * Cin, Cout).astype(_BF16)
    th = _row_tile(Ho, row_tile)
    return pl.pallas_call(
        _c3x3s2_body,
        out_shape=jax.ShapeDtypeStruct((N, Ho, Wo, Cout), _BF16),
        grid=(N, Ho // th),
        in_specs=[
            pl.BlockSpec((1, Ho + 1, 2, Wo + 1, 2 * Cin),
                         lambda n, r: (n, 0, 0, 0, 0)),
            pl.BlockSpec((12 * Cin, Cout), lambda n, r: (0, 0)),
            pl.BlockSpec((1, Cout), lambda n, r: (0, 0)),
        ],
        out_specs=pl.BlockSpec((1, th, Wo, Cout), lambda n, r: (n, r, 0, 0)),
        compiler_params=pltpu.CompilerParams(
            dimension_semantics=("parallel", "arbitrary")),
    )(xpp, w12, b.reshape(1, Cout))


def _conv3x3_s2_relu(x, w, b, row_tile=32):
    N, H, W, Cin = x.shape
    xp = jnp.pad(x, ((0, 0), (1, 1), (1, 1), (0, 0)))
    return _conv3x3_s2_relu_padded(xp, w, b, row_tile)


def _conv3x3_s2_relu_padded(xp, w, b, row_tile=32):
    N, Hp2, Wp2, Cin = xp.shape
    Ho, Wo = (Hp2 - 2) // 2, (Wp2 - 2) // 2
    xpp = xp.reshape(N, Ho + 1, 2, Wo + 1, 2 * Cin)
    return _conv3x3_s2_relu_packed(xpp, w, b, row_tile)


def _fused_level_body(*refs, th, W, Cin, C, H, topdown, emit_inner, emit_pool):
    it = iter(refs)
    fp_ref = next(it)
    c_ref = next(it) if topdown else None
    wi_ref = next(it)
    bi_ref = next(it)
    wl_ref = next(it)
    bl_ref = next(it)
    inner_ref = next(it) if emit_inner else None
    res_ref = next(it)
    pool_ref = next(it) if emit_pool else None
    pad_ref = next(it)

    r = pl.program_id(1)
    r0 = pl.multiple_of(r * th, th)

    xb = fp_ref[0, pl.ds(r0, th + 2), 1:W + 1, :].reshape((th + 2) * W, Cin)
    lat = jnp.dot(xb, wi_ref[...], preferred_element_type=jnp.float32)
    lat = (lat + bi_ref[...]).reshape(th + 2, W, C)

    mtop = jnp.where(r0 > 0, 1.0, 0.0).astype(jnp.float32)
    mbot = jnp.where(r0 + th < H, 1.0, 0.0).astype(jnp.float32)

    if topdown:
        Hc, Wc = H // 2, W // 2
        cs = c_ref[0, pl.ds(r0 // 2, th // 2), :, :].astype(jnp.float32)
        up = jnp.broadcast_to(cs.reshape(th // 2, 1, Wc, 1, C),
                              (th // 2, 2, Wc, 2, C)).reshape(th, W, C)
        mid = lat[1:th + 1] + up
        ctop = jnp.maximum(r0 // 2 - 1, 0)
        cbot = jnp.minimum(r0 // 2 + th // 2, Hc - 1)
        rt = c_ref[0, pl.ds(ctop, 1), :, :].astype(jnp.float32)
        rb = c_ref[0, pl.ds(cbot, 1), :, :].astype(jnp.float32)
        upt = jnp.broadcast_to(rt.reshape(1, Wc, 1, C),
                               (1, Wc, 2, C)).reshape(1, W, C)
        upb = jnp.broadcast_to(rb.reshape(1, Wc, 1, C),
                               (1, Wc, 2, C)).reshape(1, W, C)
        top = (lat[0:1] + upt) * mtop
        bot = (lat[th + 1:th + 2] + upb) * mbot
    else:
        mid = lat[1:th + 1]
        top = lat[0:1] * mtop
        bot = lat[th + 1:th + 2] * mbot

    if emit_inner:
        inner_ref[...] = mid.reshape(1, th, W, C).astype(inner_ref.dtype)

    zcol = jnp.zeros((th + 2, 1, C), _BF16)
    pad_ref[:, 0:1, :] = zcol
    pad_ref[:, W + 1:W + 2, :] = zcol
    pad_ref[0:1, 1:W + 1, :] = top.astype(_BF16)
    pad_ref[1:th + 1, 1:W + 1, :] = mid.astype(_BF16)
    pad_ref[th + 1:th + 2, 1:W + 1, :] = bot.astype(_BF16)

    pieces = []
    for kh in range(3):
        band = pad_ref[kh:kh + th, :, :]
        for kw in range(3):
            pieces.append(band[:, kw:kw + W, :])
    patches = jnp.concatenate(pieces, axis=-1).reshape(th * W, 9 * C)
    res = jnp.dot(patches, wl_ref[...], preferred_element_type=jnp.float32)
    res = res + bl_ref[...]
    res_ref[...] = res.reshape(1, th, W, C).astype(res_ref.dtype)

    if emit_pool:
        dec = res.reshape(th // 2, 2, W // 2, 2, C)[:, 0, :, 0, :]
        pool_ref[...] = dec.reshape(1, th // 2, W // 2, C).astype(pool_ref.dtype)


def _fused_level(featp, coarse, w_in, b_in, w_l, b_l, *,
                 emit_inner, emit_pool, row_tile=32):
    N, Hp2, Wp2, Cin = featp.shape
    H, W = Hp2 - 2, Wp2 - 2
    C = w_in.shape[1]
    th = _row_tile(H, row_tile)
    topdown = coarse is not None

    in_specs = [pl.BlockSpec((1, H + 2, W + 2, Cin), lambda n, r: (n, 0, 0, 0))]
    inputs = [featp]
    if topdown:
        in_specs.append(
            pl.BlockSpec((1, H // 2, W // 2, C), lambda n, r: (n, 0, 0, 0)))
        inputs.append(coarse)
    in_specs += [
        pl.BlockSpec((Cin, C), lambda n, r: (0, 0)),
        pl.BlockSpec((1, C), lambda n, r: (0, 0)),
        pl.BlockSpec((9 * C, C), lambda n, r: (0, 0)),
        pl.BlockSpec((1, C), lambda n, r: (0, 0)),
    ]
    inputs += [w_in.astype(_BF16), b_in.reshape(1, C),
               w_l.reshape(9 * C, C).astype(_BF16), b_l.reshape(1, C)]

    out_shapes = []
    out_specs = []
    if emit_inner:
        out_shapes.append(jax.ShapeDtypeStruct((N, H, W, C), _BF16))
        out_specs.append(pl.BlockSpec((1, th, W, C), lambda n, r: (n, r, 0, 0)))
    out_shapes.append(jax.ShapeDtypeStruct((N, H, W, C), jnp.float32))
    out_specs.append(pl.BlockSpec((1, th, W, C), lambda n, r: (n, r, 0, 0)))
    if emit_pool:
        out_shapes.append(
            jax.ShapeDtypeStruct((N, H // 2, W // 2, C), jnp.float32))
        out_specs.append(
            pl.BlockSpec((1, th // 2, W // 2, C), lambda n, r: (n, r, 0, 0)))

    body = functools.partial(
        _fused_level_body, th=th, W=W, Cin=Cin, C=C, H=H,
        topdown=topdown, emit_inner=emit_inner, emit_pool=emit_pool)
    return pl.pallas_call(
        body,
        out_shape=tuple(out_shapes),
        grid=(N, H // th),
        in_specs=in_specs,
        out_specs=tuple(out_specs),
        scratch_shapes=[pltpu.VMEM((th + 2, W + 2, C), _BF16)],
        compiler_params=pltpu.CompilerParams(
            dimension_semantics=("parallel", "arbitrary")),
    )(*inputs)


def kernel(x, bb_w0, bb_b0, bb_w1, bb_b1, bb_w2, bb_b2,
           inner_w0, inner_w1, inner_w2,
           inner_b0, inner_b1, inner_b2,
           layer_w0, layer_w1, layer_w2,
           layer_b0, layer_b1, layer_b2):
    xh = jnp.transpose(x, (0, 2, 3, 1)).astype(_BF16)
    f0 = _conv3x3_s2_relu(xh, bb_w0, bb_b0, row_tile=128)
    fp0 = jnp.pad(f0, ((0, 0), (1, 1), (1, 1), (0, 0)))
    f1 = _conv3x3_s2_relu_padded(fp0, bb_w1, bb_b1, row_tile=64)
    fp1 = jnp.pad(f1, ((0, 0), (1, 1), (1, 1), (0, 0)))
    f2 = _conv3x3_s2_relu_padded(fp1, bb_w2, bb_b2)
    fp2 = jnp.pad(f2, ((0, 0), (1, 1), (1, 1), (0, 0)))

    inner2, res2, pool = _fused_level(
        fp2, None, inner_w2, inner_b2, layer_w2, layer_b2,
        emit_inner=True, emit_pool=True)
    inner1, res1 = _fused_level(
        fp1, inner2, inner_w1, inner_b1, layer_w1, layer_b1,
        emit_inner=True, emit_pool=False)
    (res0,) = _fused_level(
        fp0, inner1, inner_w0, inner_b0, layer_w0, layer_b0,
        emit_inner=False, emit_pool=False, row_tile=16)

    out = OrderedDict()
    out["0"] = jnp.transpose(res0, (0, 3, 1, 2))
    out["1"] = jnp.transpose(res1, (0, 3, 1, 2))
    out["2"] = jnp.transpose(res2, (0, 3, 1, 2))
    out["pool"] = jnp.transpose(pool, (0, 3, 1, 2))
    return out


# speedup vs baseline: 1.0121x; 1.0121x over previous
import functools
from collections import OrderedDict

import jax
import jax.numpy as jnp
from jax.experimental import pallas as pl
from jax.experimental.pallas import tpu as pltpu

_BF16 = jnp.bfloat16


def _row_tile(total, target, quantum=8):
    if total <= target:
        return total
    cands = [c for c in range(quantum, target + 1, quantum) if total % c == 0]
    return max(cands) if cands else total


def _c3x3s2_body(xp_ref, w_ref, b_ref, o_ref):
    r = pl.program_id(1)
    _, th, Wo, Cout = o_ref.shape
    r0 = pl.multiple_of(r * th, th)
    bands = (
        xp_ref[0, pl.ds(r0, th), 0, :, :],
        xp_ref[0, pl.ds(r0, th), 1, :, :],
        xp_ref[0, pl.ds(r0 + 1, th), 0, :, :],
    )
    pieces = []
    for band in bands:
        pieces.append(band[:, 0:Wo, :])
        pieces.append(band[:, 1:Wo + 1, :])
    patches = jnp.concatenate(pieces, axis=-1)
    k = patches.shape[-1]
    acc = jnp.dot(patches.reshape(th * Wo, k), w_ref[...],
                  preferred_element_type=jnp.float32)
    acc = jnp.maximum(acc + b_ref[...], 0.0)
    o_ref[...] = acc.reshape(1, th, Wo, Cout).astype(o_ref.dtype)


def _conv3x3_s2_relu_packed(xpp, w, b, row_tile=32):
    N, Hop, _, Wop, Cin2 = xpp.shape
    Cin = Cin2 // 2
    Cout = w.shape[-1]
    Ho, Wo = Hop - 1, Wop - 1
    w4 = jnp.concatenate([w, jnp.zeros((3, 1, Cin, Cout), w.dtype)], axis=1)
    w12 = w4.reshape(12 * Cin, Cout).astype(_BF16)
    th = _row_tile(Ho, row_tile)
    return pl.pallas_call(
        _c3x3s2_body,
        out_shape=jax.ShapeDtypeStruct((N, Ho, Wo, Cout), _BF16),
        grid=(N, Ho // th),
        in_specs=[
            pl.BlockSpec((1, Ho + 1, 2, Wo + 1, 2 * Cin),
                         lambda n, r: (n, 0, 0, 0, 0)),
            pl.BlockSpec((12 * Cin, Cout), lambda n, r: (0, 0)),
            pl.BlockSpec((1, Cout), lambda n, r: (0, 0)),
        ],
        out_specs=pl.BlockSpec((1, th, Wo, Cout), lambda n, r: (n, r, 0, 0)),
        compiler_params=pltpu.CompilerParams(
            dimension_semantics=("parallel", "arbitrary")),
    )(xpp, w12, b.reshape(1, Cout))


def _conv3x3_s2_relu(x, w, b, row_tile=32):
    N, H, W, Cin = x.shape
    xp = jnp.pad(x, ((0, 0), (1, 1), (1, 1), (0, 0)))
    return _conv3x3_s2_relu_padded(xp, w, b, row_tile)


def _conv3x3_s2_relu_padded(xp, w, b, row_tile=32):
    N, Hp2, Wp2, Cin = xp.shape
    Ho, Wo = (Hp2 - 2) // 2, (Wp2 - 2) // 2
    xpp = xp.reshape(N, Ho + 1, 2, Wo + 1, 2 * Cin)
    return _conv3x3_s2_relu_packed(xpp, w, b, row_tile)


def _fused_level_body(*refs, th, W, Cin, C, H, topdown, emit_inner, emit_pool):
    it = iter(refs)
    fp_ref = next(it)
    c_ref = next(it) if topdown else None
    wi_ref = next(it)
    bi_ref = next(it)
    wl_ref = next(it)
    bl_ref = next(it)
    inner_ref = next(it) if emit_inner else None
    res_ref = next(it)
    pool_ref = next(it) if emit_pool else None
    pad_ref = next(it)

    r = pl.program_id(1)
    r0 = pl.multiple_of(r * th, th)

    xb = fp_ref[0, pl.ds(r0, th + 2), 1:W + 1, :].reshape((th + 2) * W, Cin)
    lat = jnp.dot(xb, wi_ref[...], preferred_element_type=jnp.float32)
    lat = (lat + bi_ref[...]).reshape(th + 2, W, C)

    mtop = jnp.where(r0 > 0, 1.0, 0.0).astype(jnp.float32)
    mbot = jnp.where(r0 + th < H, 1.0, 0.0).astype(jnp.float32)

    if topdown:
        Hc, Wc = H // 2, W // 2
        cs = c_ref[0, pl.ds(r0 // 2, th // 2), :, :].astype(jnp.float32)
        up = jnp.broadcast_to(cs.reshape(th // 2, 1, Wc, 1, C),
                              (th // 2, 2, Wc, 2, C)).reshape(th, W, C)
        mid = lat[1:th + 1] + up
        ctop = jnp.maximum(r0 // 2 - 1, 0)
        cbot = jnp.minimum(r0 // 2 + th // 2, Hc - 1)
        rt = c_ref[0, pl.ds(ctop, 1), :, :].astype(jnp.float32)
        rb = c_ref[0, pl.ds(cbot, 1), :, :].astype(jnp.float32)
        upt = jnp.broadcast_to(rt.reshape(1, Wc, 1, C),
                               (1, Wc, 2, C)).reshape(1, W, C)
        upb = jnp.broadcast_to(rb.reshape(1, Wc, 1, C),
                               (1, Wc, 2, C)).reshape(1, W, C)
        top = (lat[0:1] + upt) * mtop
        bot = (lat[th + 1:th + 2] + upb) * mbot
    else:
        mid = lat[1:th + 1]
        top = lat[0:1] * mtop
        bot = lat[th + 1:th + 2] * mbot

    if emit_inner:
        inner_ref[...] = mid.reshape(1, th, W, C).astype(inner_ref.dtype)

    zcol = jnp.zeros((th + 2, 1, C), _BF16)
    pad_ref[:, 0:1, :] = zcol
    pad_ref[:, W + 1:W + 2, :] = zcol
    pad_ref[0:1, 1:W + 1, :] = top.astype(_BF16)
    pad_ref[1:th + 1, 1:W + 1, :] = mid.astype(_BF16)
    pad_ref[th + 1:th + 2, 1:W + 1, :] = bot.astype(_BF16)

    pieces = []
    for kh in range(3):
        band = pad_ref[kh:kh + th, :, :]
        for kw in range(3):
            pieces.append(band[:, kw:kw + W, :])
    patches = jnp.concatenate(pieces, axis=-1).reshape(th * W, 9 * C)
    res = jnp.dot(patches, wl_ref[...], preferred_element_type=jnp.float32)
    res = res + bl_ref[...]
    res_ref[...] = res.reshape(1, th, W, C).astype(res_ref.dtype)

    if emit_pool:
        dec = res.reshape(th // 2, 2, W // 2, 2, C)[:, 0, :, 0, :]
        pool_ref[...] = dec.reshape(1, th // 2, W // 2, C).astype(pool_ref.dtype)


def _fused_level(featp, coarse, w_in, b_in, w_l, b_l, *,
                 emit_inner, emit_pool, row_tile=32):
    N, Hp2, Wp2, Cin = featp.shape
    H, W = Hp2 - 2, Wp2 - 2
    C = w_in.shape[1]
    th = _row_tile(H, row_tile)
    topdown = coarse is not None

    in_specs = [pl.BlockSpec((1, H + 2, W + 2, Cin), lambda n, r: (n, 0, 0, 0))]
    inputs = [featp]
    if topdown:
        in_specs.append(
            pl.BlockSpec((1, H // 2, W // 2, C), lambda n, r: (n, 0, 0, 0)))
        inputs.append(coarse)
    in_specs += [
        pl.BlockSpec((Cin, C), lambda n, r: (0, 0)),
        pl.BlockSpec((1, C), lambda n, r: (0, 0)),
        pl.BlockSpec((9 * C, C), lambda n, r: (0, 0)),
        pl.BlockSpec((1, C), lambda n, r: (0, 0)),
    ]
    inputs += [w_in.astype(_BF16), b_in.reshape(1, C),
               w_l.reshape(9 * C, C).astype(_BF16), b_l.reshape(1, C)]

    out_shapes = []
    out_specs = []
    if emit_inner:
        out_shapes.append(jax.ShapeDtypeStruct((N, H, W, C), _BF16))
        out_specs.append(pl.BlockSpec((1, th, W, C), lambda n, r: (n, r, 0, 0)))
    out_shapes.append(jax.ShapeDtypeStruct((N, H, W, C), jnp.float32))
    out_specs.append(pl.BlockSpec((1, th, W, C), lambda n, r: (n, r, 0, 0)))
    if emit_pool:
        out_shapes.append(
            jax.ShapeDtypeStruct((N, H // 2, W // 2, C), jnp.float32))
        out_specs.append(
            pl.BlockSpec((1, th // 2, W // 2, C), lambda n, r: (n, r, 0, 0)))

    body = functools.partial(
        _fused_level_body, th=th, W=W, Cin=Cin, C=C, H=H,
        topdown=topdown, emit_inner=emit_inner, emit_pool=emit_pool)
    return pl.pallas_call(
        body,
        out_shape=tuple(out_shapes),
        grid=(N, H // th),
        in_specs=in_specs,
        out_specs=tuple(out_specs),
        scratch_shapes=[pltpu.VMEM((th + 2, W + 2, C), _BF16)],
        compiler_params=pltpu.CompilerParams(
            dimension_semantics=("parallel", "arbitrary")),
    )(*inputs)


def kernel(x, bb_w0, bb_b0, bb_w1, bb_b1, bb_w2, bb_b2,
           inner_w0, inner_w1, inner_w2,
           inner_b0, inner_b1, inner_b2,
           layer_w0, layer_w1, layer_w2,
           layer_b0, layer_b1, layer_b2):
    xh = jnp.transpose(x, (0, 2, 3, 1)).astype(_BF16)
    f0 = _conv3x3_s2_relu(xh, bb_w0, bb_b0, row_tile=128)
    fp0 = jnp.pad(f0, ((0, 0), (1, 1), (1, 1), (0, 0)))
    f1 = _conv3x3_s2_relu_padded(fp0, bb_w1, bb_b1, row_tile=64)
    fp1 = jnp.pad(f1, ((0, 0), (1, 1), (1, 1), (0, 0)))
    f2 = _conv3x3_s2_relu_padded(fp1, bb_w2, bb_b2)
    fp2 = jnp.pad(f2, ((0, 0), (1, 1), (1, 1), (0, 0)))

    inner2, res2, pool = _fused_level(
        fp2, None, inner_w2, inner_b2, layer_w2, layer_b2,
        emit_inner=True, emit_pool=True)
    inner1, res1 = _fused_level(
        fp1, inner2, inner_w1, inner_b1, layer_w1, layer_b1,
        emit_inner=True, emit_pool=False)
    (res0,) = _fused_level(
        fp0, inner1, inner_w0, inner_b0, layer_w0, layer_b0,
        emit_inner=False, emit_pool=False)

    out = OrderedDict()
    out["0"] = jnp.transpose(res0, (0, 3, 1, 2))
    out["1"] = jnp.transpose(res1, (0, 3, 1, 2))
    out["2"] = jnp.transpose(res2, (0, 3, 1, 2))
    out["pool"] = jnp.transpose(pool, (0, 3, 1, 2))
    return out
